# baseline (speedup 1.0000x reference)
import functools

import jax
import jax.numpy as jnp
from jax import lax
from jax.experimental import pallas as pl
from jax.experimental.pallas import tpu as pltpu
from jax.experimental.pallas import tpu_sc as plsc

VOCAB = 100000
EMBED_DIM = 64
BATCH = 16384

NUM_CORES = 2
NUM_SUBCORES = 16
NUM_WORKERS = NUM_CORES * NUM_SUBCORES
NUM_PASSES = EMBED_DIM // NUM_WORKERS
ICHUNK = 4096
NCHUNKS = BATCH // ICHUNK
LANES = 16
UNROLL = 4
STEPS = ICHUNK // (LANES * UNROLL)

_mesh = plsc.VectorSubcoreMesh(core_axis_name="c", subcore_axis_name="s")


@functools.partial(
    pl.kernel,
    mesh=_mesh,
    out_type=jax.ShapeDtypeStruct((EMBED_DIM, BATCH), jnp.float32),
    scratch_types=[
        pltpu.VMEM((VOCAB,), jnp.float32),
        pltpu.VMEM((BATCH,), jnp.int32),
        pltpu.VMEM((2, ICHUNK), jnp.float32),
        pltpu.SemaphoreType.DMA,
        pltpu.SemaphoreType.DMA,
        pltpu.SemaphoreType.DMA,
    ],
    compiler_params=pltpu.CompilerParams(needs_layout_passes=False),
)
def _gather_kernel(idx_hbm, tabT_hbm, outT_hbm, row_v, idx_v, oc_v, sem, osem0, osem1):
    wid = lax.axis_index("s") * NUM_CORES + lax.axis_index("c")
    idx_cp = pltpu.async_copy(idx_hbm, idx_v, sem)
    row_cp = pltpu.async_copy(tabT_hbm.at[wid, :], row_v, sem)
    idx_cp.wait()
    row_cp.wait()
    osems = (osem0, osem1)
    out_cps = [None, None]
    for p in range(NUM_PASSES):
        f = p * NUM_WORKERS + wid
        for c in range(NCHUNKS):
            buf = (p * NCHUNKS + c) % 2
            if out_cps[buf] is not None:
                out_cps[buf].wait()

            def body(j, _):
                for u in range(UNROLL):
                    off = c * ICHUNK + j * LANES * UNROLL + u * LANES
                    iv = idx_v[pl.ds(off, LANES)]
                    oc_v[buf, pl.ds(j * LANES * UNROLL + u * LANES, LANES)] = (
                        plsc.load_gather(row_v, [iv])
                    )
                return 0

            lax.fori_loop(0, STEPS, body, 0)
            out_cps[buf] = pltpu.async_copy(
                oc_v.at[buf], outT_hbm.at[f, pl.ds(c * ICHUNK, ICHUNK)], osems[buf]
            )
        if p + 1 < NUM_PASSES:
            pltpu.sync_copy(tabT_hbm.at[p * NUM_WORKERS + NUM_WORKERS + wid, :], row_v)
    for cp in out_cps:
        if cp is not None:
            cp.wait()


def kernel(indices, table):
    outT = _gather_kernel(indices.astype(jnp.int32), table.T)
    return outT.T


# speedup vs baseline: 1.4246x; 1.4246x over previous
import functools

import jax
import jax.numpy as jnp
from jax import lax
from jax.experimental import pallas as pl
from jax.experimental.pallas import tpu as pltpu
from jax.experimental.pallas import tpu_sc as plsc

VOCAB = 100000
EMBED_DIM = 64
BATCH = 16384

NUM_CORES = 2
NUM_SUBCORES = 16
NUM_WORKERS = NUM_CORES * NUM_SUBCORES
NUM_PASSES = EMBED_DIM // NUM_WORKERS
ICHUNK = 8192
NCHUNKS = BATCH // ICHUNK
LANES = 16

_mesh = plsc.VectorSubcoreMesh(core_axis_name="c", subcore_axis_name="s")


@functools.partial(
    pl.kernel,
    mesh=_mesh,
    out_type=jax.ShapeDtypeStruct((EMBED_DIM, BATCH), jnp.float32),
    scratch_types=[
        pltpu.VMEM((VOCAB,), jnp.float32),
        pltpu.VMEM((BATCH,), jnp.int32),
        pltpu.VMEM((ICHUNK,), jnp.float32),
        pltpu.SemaphoreType.DMA,
    ],
    compiler_params=pltpu.CompilerParams(needs_layout_passes=False),
)
def _gather_kernel(idx_hbm, tabT_hbm, outT_hbm, row_v, idx_v, oc_v, sem):
    wid = lax.axis_index("s") * NUM_CORES + lax.axis_index("c")
    idx_cp = pltpu.async_copy(idx_hbm, idx_v, sem)
    row_cp = pltpu.async_copy(tabT_hbm.at[wid, :], row_v, sem)
    idx_cp.wait()
    row_cp.wait()
    for p in range(NUM_PASSES):
        f = p * NUM_WORKERS + wid
        for c in range(NCHUNKS):
            @plsc.parallel_loop(0, ICHUNK // LANES, unroll=8)
            def body(j):
                iv = idx_v[pl.ds(c * ICHUNK + j * LANES, LANES)]
                oc_v[pl.ds(j * LANES, LANES)] = plsc.load_gather(row_v, [iv])

            pltpu.sync_copy(oc_v, outT_hbm.at[f, pl.ds(c * ICHUNK, ICHUNK)])
        if p + 1 < NUM_PASSES:
            pltpu.sync_copy(tabT_hbm.at[(p + 1) * NUM_WORKERS + wid, :], row_v)


def kernel(indices, table):
    outT = _gather_kernel(indices.astype(jnp.int32), table.T)
    return outT.T
